# baseline (speedup 1.0000x reference)
import functools

import jax
import jax.numpy as jnp
from jax import lax
from jax.experimental import pallas as pl
from jax.experimental.pallas import tpu as pltpu
from jax.experimental.pallas import tpu_sc as plsc

L = 12
H = 16
HS = 64
IN1 = 1024
IN3 = 4096
_TOP = 0x7FFFFFFF


def _sq_reduce_body(w_ref, out_ref):
    x = w_ref[0]
    out_ref[0, 0, :] = jnp.sum(x * x, axis=0)


def _tc_reduce_one(W, cols):
    return pl.pallas_call(
        _sq_reduce_body,
        grid=(L, 4),
        in_specs=[pl.BlockSpec((1, IN1, cols // 4), lambda l, c: (l, 0, c))],
        out_specs=pl.BlockSpec((1, 1, cols // 4), lambda l, c: (l, 0, c)),
        out_shape=jax.ShapeDtypeStruct((L, 1, cols), jnp.float32),
    )(W)


def _sc_big_body(sc3_hbm, pi_hbm, pf_hbm, out_hbm, buf4k, pi_v, pf_v, outv):
    cid = lax.axis_index("c")
    sid = lax.axis_index("s")
    wid = cid * 16 + sid
    lane = lax.iota(jnp.int32, 16)
    zeros_f = jnp.zeros((16,), jnp.float32)
    zeros_i = jnp.zeros((16,), jnp.int32)
    outv[...] = zeros_f

    @pl.when((cid == 0) & (sid < L))
    def _big():
        pltpu.sync_copy(sc3_hbm.at[sid], buf4k)
        pltpu.sync_copy(pi_hbm.at[sid], pi_v)
        pltpu.sync_copy(pf_hbm.at[sid], pf_v)
        k2 = pi_v[pl.ds(0, 16)][0]
        y1 = pf_v[pl.ds(0, 16)][0]
        kvec = jnp.full((16,), 1, jnp.int32) * k2

        @plsc.parallel_loop(0, IN3 // 16, unroll=8,
                            carry=(zeros_f, zeros_i,
                                   jnp.full((16,), _TOP, jnp.int32)))
        def _init(i, c):
            tv, mx, mn = c
            wb = buf4k[pl.ds(i * 16, 16)]
            return (tv + lax.bitcast_convert_type(wb, jnp.float32),
                    jnp.maximum(mx, wb), jnp.minimum(mn, wb))

        totv, mxv, mnv = _init
        tot_s = jnp.sum(totv)
        lo0 = jnp.full((16,), 1, jnp.int32) * jnp.min(mnv)
        hi0 = jnp.full((16,), 1, jnp.int32) * jnp.max(mxv)

        def unconverged(carry):
            lo, hi = carry
            return (hi - lo)[0] > 0

        def qpass(carry):
            lo, hi = carry
            mid = lo + lax.shift_right_logical(hi - lo, 1)

            @plsc.parallel_loop(0, IN3 // 16, unroll=8, carry=zeros_i)
            def _cnt(i, cnt):
                wb = buf4k[pl.ds(i * 16, 16)]
                return cnt + plsc.all_reduce_population_count(wb <= mid)

            ge = _cnt >= kvec
            return jnp.where(ge, lo, mid + 1), jnp.where(ge, mid, hi)

        lo, _ = lax.while_loop(unconverged, qpass, (lo0, hi0))
        t = lax.bitcast_convert_type(lo, jnp.float32)

        @plsc.parallel_loop(0, IN3 // 16, unroll=8, carry=(zeros_i, zeros_f))
        def _fin(i, c):
            clt, slt = c
            wb = buf4k[pl.ds(i * 16, 16)]
            m = wb < lo
            return (clt + plsc.all_reduce_population_count(m),
                    slt + jnp.where(
                        m, lax.bitcast_convert_type(wb, jnp.float32), 0.0))

        cltv, sltv = _fin
        res = jnp.where(
            k2 >= IN3, tot_s,
            jnp.where(k2 <= 0, 0.0,
                      jnp.sum(sltv)
                      + (k2 - jnp.max(cltv)).astype(jnp.float32)
                      * jnp.max(t)))
        outv[...] = jnp.where(lane == 0, y1 * res, 0.0)

    pltpu.sync_copy(outv, out_hbm.at[wid])


def _sc_small_body(colsq_hbm, pi_hbm, pf_hbm, out_hbm, buf1k, pi_v, pf_v,
                   outv):
    cid = lax.axis_index("c")
    sid = lax.axis_index("s")
    wid = cid * 16 + sid
    lane = lax.iota(jnp.int32, 16)
    zeros_f = jnp.zeros((16,), jnp.float32)
    zeros_i = jnp.zeros((16,), jnp.int32)
    top_v = jnp.full((16,), _TOP, jnp.int32)
    outv[...] = zeros_f

    @pl.when((cid == 0) & (sid < L))
    def _small():
        pltpu.sync_copy(colsq_hbm.at[sid], buf1k)
        pltpu.sync_copy(pi_hbm.at[sid], pi_v)
        pltpu.sync_copy(pf_hbm.at[sid], pf_v)
        kr = pi_v[pl.ds(0, 16)]
        pvec = pf_v[pl.ds(0, 16)]
        k1 = pi_v[pl.ds(16, 16)][0]
        y0 = pf_v[pl.ds(16, 16)][0]
        base = lane * HS

        @plsc.parallel_loop(0, HS, unroll=8, carry=(zeros_f, zeros_i, top_v))
        def _init(j, c):
            tv, mx, mn = c
            wb = plsc.load_gather(buf1k, [base + j])
            return (tv + lax.bitcast_convert_type(wb, jnp.float32),
                    jnp.maximum(mx, wb), jnp.minimum(mn, wb))

        tot, mxv, mnv = _init

        def unconverged(carry):
            lo, hi = carry
            return jnp.max(hi - lo) > 0

        def qpass(carry):
            lo, hi = carry
            mid = lo + lax.shift_right_logical(hi - lo, 1)

            @plsc.parallel_loop(0, HS, unroll=8, carry=zeros_i)
            def _cnt(j, cnt):
                wb = plsc.load_gather(buf1k, [base + j])
                return cnt + jnp.where(wb <= mid, 1, 0).astype(jnp.int32)

            ge = _cnt >= kr
            return jnp.where(ge, lo, mid + 1), jnp.where(ge, mid, hi)

        lo, _ = lax.while_loop(unconverged, qpass, (mnv, mxv))
        t = lax.bitcast_convert_type(lo, jnp.float32)

        @plsc.parallel_loop(0, HS, unroll=8, carry=(zeros_i, zeros_f))
        def _fin(j, c):
            clt, slt = c
            wb = plsc.load_gather(buf1k, [base + j])
            m = wb < lo
            return (clt + jnp.where(m, 1, 0).astype(jnp.int32),
                    slt + jnp.where(
                        m, lax.bitcast_convert_type(wb, jnp.float32), 0.0))

        cltv, sltv = _fin
        res = jnp.where(
            kr >= HS, tot,
            jnp.where(kr <= 0, 0.0,
                      sltv + (kr - cltv).astype(jnp.float32) * t))
        rres = jnp.sum(pvec * res)
        sk, _ = plsc.sort_key_val(tot, tot)
        a_val = jnp.sum(jnp.where(lane < k1, sk, 0.0))
        outv[...] = jnp.where(lane == 0, y0 * a_val + rres, 0.0)

    pltpu.sync_copy(outv, out_hbm.at[wid])


def _sc_big(sc3, pi, pf):
    mesh = plsc.VectorSubcoreMesh(core_axis_name="c", subcore_axis_name="s")
    run = functools.partial(
        pl.kernel,
        mesh=mesh,
        compiler_params=pltpu.CompilerParams(needs_layout_passes=False),
        out_type=jax.ShapeDtypeStruct((32, 16), jnp.float32),
        scratch_types=[
            pltpu.VMEM((IN3,), jnp.int32),
            pltpu.VMEM((16,), jnp.int32),
            pltpu.VMEM((16,), jnp.float32),
            pltpu.VMEM((16,), jnp.float32),
        ],
    )(_sc_big_body)
    return run(sc3, pi, pf)


def _sc_small(colsq, pi, pf):
    mesh = plsc.VectorSubcoreMesh(core_axis_name="c", subcore_axis_name="s")
    run = functools.partial(
        pl.kernel,
        mesh=mesh,
        compiler_params=pltpu.CompilerParams(needs_layout_passes=False),
        out_type=jax.ShapeDtypeStruct((32, 16), jnp.float32),
        scratch_types=[
            pltpu.VMEM((IN1,), jnp.int32),
            pltpu.VMEM((32,), jnp.int32),
            pltpu.VMEM((32,), jnp.float32),
            pltpu.VMEM((16,), jnp.float32),
        ],
    )(_sc_small_body)
    return run(colsq, pi, pf)


def kernel(W1, W3, s, r, y, p):
    sc33 = _tc_reduce_one(W3, IN3)
    colsq3 = _tc_reduce_one(W1, IN1)
    colsq = lax.bitcast_convert_type(colsq3.reshape(L, IN1), jnp.int32)
    sc3 = lax.bitcast_convert_type(sc33.reshape(L, IN3), jnp.int32)
    k1 = jnp.ceil(s[:, 0]).astype(jnp.int32)
    k2 = jnp.ceil(s[:, 1]).astype(jnp.int32)
    kr = jnp.ceil(r).astype(jnp.int32)
    pad_i = jnp.zeros((L, 15), jnp.int32)
    pad_f = jnp.zeros((L, 15), jnp.float32)
    pi_b = jnp.concatenate([k2[:, None], pad_i], axis=1)
    pf_b = jnp.concatenate([y[:, 1:2], pad_f], axis=1)
    pi_s = jnp.concatenate(
        [kr, k1[:, None], jnp.zeros((L, 15), jnp.int32)], axis=1)
    pf_s = jnp.concatenate(
        [p, y[:, 0:1], jnp.zeros((L, 15), jnp.float32)], axis=1)
    parts_b = _sc_big(sc3, pi_b, pf_b)
    parts_s = _sc_small(colsq, pi_s, pf_s)
    return jnp.sum(parts_b) + jnp.sum(parts_s)


# speedup vs baseline: 1.2250x; 1.2250x over previous
import functools

import jax
import jax.numpy as jnp
from jax import lax
from jax.experimental import pallas as pl
from jax.experimental.pallas import tpu as pltpu
from jax.experimental.pallas import tpu_sc as plsc

L = 12
H = 16
HS = 64
IN1 = 1024
IN3 = 4096
_TOP = 0x7FFFFFFF


def _sq_reduce_body(w1_ref, w3_ref, colsq_ref, sc3_ref):
    x1 = w1_ref[0]
    colsq_ref[0, 0, :] = jnp.sum(x1 * x1, axis=0)
    x3 = w3_ref[0]
    sc3_ref[0, 0, :] = jnp.sum(x3 * x3, axis=0)


def _tc_reduce(W1, W3):
    return pl.pallas_call(
        _sq_reduce_body,
        grid=(L, 4),
        in_specs=[
            pl.BlockSpec((1, IN1, IN1 // 4), lambda l, c: (l, 0, c)),
            pl.BlockSpec((1, IN1, IN3 // 4), lambda l, c: (l, 0, c)),
        ],
        out_specs=[
            pl.BlockSpec((1, 1, IN1 // 4), lambda l, c: (l, 0, c)),
            pl.BlockSpec((1, 1, IN3 // 4), lambda l, c: (l, 0, c)),
        ],
        out_shape=[
            jax.ShapeDtypeStruct((L, 1, IN1), jnp.float32),
            jax.ShapeDtypeStruct((L, 1, IN3), jnp.float32),
        ],
    )(W1, W3)


def _sc_body(colsq_hbm, sc3_hbm, pi_hbm, pf_hbm, out_hbm,
             bufa, bufb, buf1k, pi_v, pf_v, outv):
    cid = lax.axis_index("c")
    sid = lax.axis_index("s")
    wid = cid * 16 + sid
    lane = lax.iota(jnp.int32, 16)
    zeros_f = jnp.zeros((16,), jnp.float32)
    zeros_i = jnp.zeros((16,), jnp.int32)
    top_v = jnp.full((16,), _TOP, jnp.int32)
    outv[...] = zeros_f

    @pl.when((cid == 0) & (sid < L))
    def _big():
        pltpu.sync_copy(sc3_hbm.at[sid], bufa.at[pl.ds(0, IN3)])
        pltpu.sync_copy(pi_hbm.at[sid], pi_v)
        pltpu.sync_copy(pf_hbm.at[sid], pf_v)
        k2 = pi_v[pl.ds(16, 16)][1]
        y1 = pf_v[pl.ds(16, 16)][1]

        @plsc.parallel_loop(0, IN3 // 16, unroll=8,
                            carry=(zeros_f, zeros_i,
                                   jnp.full((16,), _TOP, jnp.int32)))
        def _init(i, c):
            tv, mx, mn = c
            wb = bufa[pl.ds(i * 16, 16)]
            return (tv + lax.bitcast_convert_type(wb, jnp.float32),
                    jnp.maximum(mx, wb), jnp.minimum(mn, wb))

        totv, mxv, mnv = _init
        tot_s = jnp.sum(totv)
        lo0 = jnp.min(mnv)
        hi0 = jnp.max(mxv)

        def live(c):
            n, k_rem, lo, hi, bs = c
            return (n > 16) & (lo < hi)

        def qpass(c):
            n, k_rem, lo, hi, bs = c
            mid = lo + lax.shift_right_logical(hi - lo, 1)
            nch = (n + 15) // 16

            def sweep(i, sc):
                off_l, off_h, sum_l = sc
                wb = bufa[pl.ds(i * 16, 16)]
                valid = (i * 16 + lane) < n
                m_lo = valid & (wb <= mid)
                m_hi = valid & (wb > mid)
                c_lo = jnp.max(plsc.all_reduce_population_count(m_lo))
                c_hi = jnp.max(plsc.all_reduce_population_count(m_hi))
                plsc.store_compressed(bufa.at[pl.ds(off_l, 16)], wb,
                                      mask=m_lo)
                plsc.store_compressed(bufb.at[pl.ds(off_h, 16)], wb,
                                      mask=m_hi)
                return (off_l + c_lo, off_h + c_hi,
                        sum_l + jnp.sum(jnp.where(
                            m_lo, lax.bitcast_convert_type(wb, jnp.float32),
                            0.0)))

            c_low, c_high, s_low = lax.fori_loop(
                0, nch, sweep, (jnp.int32(0), jnp.int32(0), jnp.float32(0.0)))
            keep_low = k_rem <= c_low

            @pl.when(jnp.logical_not(keep_low))
            def _copy_back():
                def cpy(i, _):
                    bufa[pl.ds(i * 16, 16)] = bufb[pl.ds(i * 16, 16)]
                    return 0
                lax.fori_loop(0, (c_high + 15) // 16, cpy, 0)

            return (jnp.where(keep_low, c_low, c_high),
                    jnp.where(keep_low, k_rem, k_rem - c_low),
                    jnp.where(keep_low, lo, mid + 1),
                    jnp.where(keep_low, mid, hi),
                    jnp.where(keep_low, bs, bs + s_low))

        kk = jnp.clip(k2, 0, IN3)
        n_f, k_f, lo_f, hi_f, bs_f = lax.while_loop(
            live, qpass, (jnp.int32(IN3), kk, lo0, hi0, jnp.float32(0.0)))

        wb = bufa[pl.ds(0, 16)]
        valid = lane < jnp.minimum(n_f, 16)
        sk, _, _ = plsc.sort_key_val(wb, wb, mask=valid)
        skf = jnp.where(valid, lax.bitcast_convert_type(sk, jnp.float32), 0.0)
        cs = plsc.cumsum(skf)
        sel = jnp.sum(jnp.where(lane == k_f - 1, cs, 0.0))
        tied = bs_f + k_f.astype(jnp.float32) * lax.bitcast_convert_type(
            lo_f, jnp.float32)
        fin = jnp.where(n_f > 16, tied, bs_f + sel)
        res = jnp.where(k2 >= IN3, tot_s, jnp.where(k2 <= 0, 0.0, fin))
        outv[...] = jnp.where(lane == 0, y1 * res, 0.0)

    @pl.when((cid == 1) & (sid < L))
    def _small():
        pltpu.sync_copy(colsq_hbm.at[sid], buf1k)
        pltpu.sync_copy(pi_hbm.at[sid], pi_v)
        pltpu.sync_copy(pf_hbm.at[sid], pf_v)
        kr = pi_v[pl.ds(0, 16)]
        pvec = pf_v[pl.ds(0, 16)]
        k1 = pi_v[pl.ds(16, 16)][0]
        y0 = pf_v[pl.ds(16, 16)][0]
        base = lane * HS

        @plsc.parallel_loop(0, HS, unroll=8, carry=(zeros_f, zeros_i, top_v))
        def _init(j, c):
            tv, mx, mn = c
            wb = plsc.load_gather(buf1k, [base + j])
            return (tv + lax.bitcast_convert_type(wb, jnp.float32),
                    jnp.maximum(mx, wb), jnp.minimum(mn, wb))

        tot, mxv, mnv = _init

        def unconverged(carry):
            lo, hi = carry
            return jnp.max(hi - lo) > 0

        def qpass(carry):
            lo, hi = carry
            mid = lo + lax.shift_right_logical(hi - lo, 1)

            @plsc.parallel_loop(0, HS, unroll=8, carry=zeros_i)
            def _cnt(j, cnt):
                wb = plsc.load_gather(buf1k, [base + j])
                return cnt + jnp.where(wb <= mid, 1, 0).astype(jnp.int32)

            ge = _cnt >= kr
            return jnp.where(ge, lo, mid + 1), jnp.where(ge, mid, hi)

        lo, _ = lax.while_loop(unconverged, qpass, (mnv, mxv))
        t = lax.bitcast_convert_type(lo, jnp.float32)

        @plsc.parallel_loop(0, HS, unroll=8, carry=(zeros_i, zeros_f))
        def _fin(j, c):
            clt, slt = c
            wb = plsc.load_gather(buf1k, [base + j])
            m = wb < lo
            return (clt + jnp.where(m, 1, 0).astype(jnp.int32),
                    slt + jnp.where(
                        m, lax.bitcast_convert_type(wb, jnp.float32), 0.0))

        cltv, sltv = _fin
        res = jnp.where(
            kr >= HS, tot,
            jnp.where(kr <= 0, 0.0,
                      sltv + (kr - cltv).astype(jnp.float32) * t))
        rres = jnp.sum(pvec * res)
        sk, _ = plsc.sort_key_val(tot, tot)
        a_val = jnp.sum(jnp.where(lane < k1, sk, 0.0))
        outv[...] = jnp.where(lane == 0, y0 * a_val + rres, 0.0)

    pltpu.sync_copy(outv, out_hbm.at[wid])


def _sc_select(colsq, sc3, pi, pf):
    mesh = plsc.VectorSubcoreMesh(core_axis_name="c", subcore_axis_name="s")
    run = functools.partial(
        pl.kernel,
        mesh=mesh,
        compiler_params=pltpu.CompilerParams(needs_layout_passes=False),
        out_type=jax.ShapeDtypeStruct((32, 16), jnp.float32),
        scratch_types=[
            pltpu.VMEM((IN3 + 16,), jnp.int32),
            pltpu.VMEM((IN3 + 16,), jnp.int32),
            pltpu.VMEM((IN1,), jnp.int32),
            pltpu.VMEM((32,), jnp.int32),
            pltpu.VMEM((32,), jnp.float32),
            pltpu.VMEM((16,), jnp.float32),
        ],
    )(_sc_body)
    return run(colsq, sc3, pi, pf)


def kernel(W1, W3, s, r, y, p):
    colsq3, sc33 = _tc_reduce(W1, W3)
    colsq = lax.bitcast_convert_type(colsq3.reshape(L, IN1), jnp.int32)
    sc3 = lax.bitcast_convert_type(sc33.reshape(L, IN3), jnp.int32)
    k1 = jnp.ceil(s[:, 0]).astype(jnp.int32)
    k2 = jnp.ceil(s[:, 1]).astype(jnp.int32)
    kr = jnp.ceil(r).astype(jnp.int32)
    pi = jnp.concatenate(
        [kr, k1[:, None], k2[:, None], jnp.zeros((L, 14), jnp.int32)], axis=1)
    pf = jnp.concatenate([p, y, jnp.zeros((L, 14), jnp.float32)], axis=1)
    parts = _sc_select(colsq, sc3, pi, pf)
    return jnp.sum(parts)


# speedup vs baseline: 1.2279x; 1.0024x over previous
import functools

import jax
import jax.numpy as jnp
from jax import lax
from jax.experimental import pallas as pl
from jax.experimental.pallas import tpu as pltpu
from jax.experimental.pallas import tpu_sc as plsc

L = 12
H = 16
HS = 64
IN1 = 1024
IN3 = 4096
_TOP = 0x7FFFFFFF


def _sq_reduce_body(w1_ref, w3_ref, colsq_ref, sc3_ref):
    x1 = w1_ref[0]
    colsq_ref[0, 0, :] = jnp.sum(x1 * x1, axis=0)
    x3 = w3_ref[0]
    sc3_ref[0, 0, :] = jnp.sum(x3 * x3, axis=0)


def _tc_reduce(W1, W3):
    return pl.pallas_call(
        _sq_reduce_body,
        grid=(L, 4),
        in_specs=[
            pl.BlockSpec((1, IN1, IN1 // 4), lambda l, c: (l, 0, c)),
            pl.BlockSpec((1, IN1, IN3 // 4), lambda l, c: (l, 0, c)),
        ],
        out_specs=[
            pl.BlockSpec((1, 1, IN1 // 4), lambda l, c: (l, 0, c)),
            pl.BlockSpec((1, 1, IN3 // 4), lambda l, c: (l, 0, c)),
        ],
        out_shape=[
            jax.ShapeDtypeStruct((L, 1, IN1), jnp.float32),
            jax.ShapeDtypeStruct((L, 1, IN3), jnp.float32),
        ],
    )(W1, W3)


def _sc_body(colsq_hbm, sc3_hbm, pi_hbm, pf_hbm, out_hbm,
             bufa, bufb, buf1k, pi_v, pf_v, outv):
    cid = lax.axis_index("c")
    sid = lax.axis_index("s")
    wid = cid * 16 + sid
    lane = lax.iota(jnp.int32, 16)
    zeros_f = jnp.zeros((16,), jnp.float32)
    zeros_i = jnp.zeros((16,), jnp.int32)
    top_v = jnp.full((16,), _TOP, jnp.int32)
    outv[...] = zeros_f

    @pl.when((cid == 0) & (sid < L))
    def _big():
        pltpu.sync_copy(sc3_hbm.at[sid], bufa.at[pl.ds(0, IN3)])
        pltpu.sync_copy(pi_hbm.at[sid], pi_v)
        pltpu.sync_copy(pf_hbm.at[sid], pf_v)
        k2 = pi_v[pl.ds(16, 16)][1]
        y1 = pf_v[pl.ds(16, 16)][1]

        @plsc.parallel_loop(0, IN3 // 16, unroll=8,
                            carry=(zeros_f, zeros_i,
                                   jnp.full((16,), _TOP, jnp.int32)))
        def _init(i, c):
            tv, mx, mn = c
            wb = bufa[pl.ds(i * 16, 16)]
            return (tv + lax.bitcast_convert_type(wb, jnp.float32),
                    jnp.maximum(mx, wb), jnp.minimum(mn, wb))

        totv, mxv, mnv = _init
        tot_s = jnp.sum(totv)
        lo0 = jnp.min(mnv)
        hi0 = jnp.max(mxv)

        def live(c):
            n, k_rem, lo, hi, bs = c
            return (n > 16) & (lo < hi)

        def qpass(c):
            n, k_rem, lo, hi, bs = c
            mid = lo + lax.shift_right_logical(hi - lo, 1)
            nch = (n + 15) // 16

            def sweep(i, sc):
                off_l, off_h, sum_l = sc
                wb = bufa[pl.ds(i * 16, 16)]
                valid = (i * 16 + lane) < n
                m_lo = valid & (wb <= mid)
                m_hi = valid & (wb > mid)
                c_lo = jnp.max(plsc.all_reduce_population_count(m_lo))
                c_hi = jnp.max(plsc.all_reduce_population_count(m_hi))
                plsc.store_compressed(bufa.at[pl.ds(off_l, 16)], wb,
                                      mask=m_lo)
                plsc.store_compressed(bufb.at[pl.ds(off_h, 16)], wb,
                                      mask=m_hi)
                return (off_l + c_lo, off_h + c_hi,
                        sum_l + jnp.sum(jnp.where(
                            m_lo, lax.bitcast_convert_type(wb, jnp.float32),
                            0.0)))

            c_low, c_high, s_low = lax.fori_loop(
                0, nch, sweep, (jnp.int32(0), jnp.int32(0), jnp.float32(0.0)))
            keep_low = k_rem <= c_low

            @pl.when(jnp.logical_not(keep_low))
            def _copy_back():
                def cpy(i, _):
                    bufa[pl.ds(i * 16, 16)] = bufb[pl.ds(i * 16, 16)]
                    return 0
                lax.fori_loop(0, (c_high + 15) // 16, cpy, 0)

            return (jnp.where(keep_low, c_low, c_high),
                    jnp.where(keep_low, k_rem, k_rem - c_low),
                    jnp.where(keep_low, lo, mid + 1),
                    jnp.where(keep_low, mid, hi),
                    jnp.where(keep_low, bs, bs + s_low))

        kk = jnp.clip(k2, 0, IN3)
        n_f, k_f, lo_f, hi_f, bs_f = lax.while_loop(
            live, qpass, (jnp.int32(IN3), kk, lo0, hi0, jnp.float32(0.0)))

        wb = bufa[pl.ds(0, 16)]
        valid = lane < jnp.minimum(n_f, 16)
        sk, _, _ = plsc.sort_key_val(wb, wb, mask=valid)
        skf = jnp.where(valid, lax.bitcast_convert_type(sk, jnp.float32), 0.0)
        cs = plsc.cumsum(skf)
        sel = jnp.sum(jnp.where(lane == k_f - 1, cs, 0.0))
        tied = bs_f + k_f.astype(jnp.float32) * lax.bitcast_convert_type(
            lo_f, jnp.float32)
        fin = jnp.where(n_f > 16, tied, bs_f + sel)
        res = jnp.where(k2 >= IN3, tot_s, jnp.where(k2 <= 0, 0.0, fin))
        outv[...] = jnp.where(lane == 0, y1 * res, 0.0)

    @pl.when((cid == 1) & (sid < L))
    def _small():
        pltpu.sync_copy(colsq_hbm.at[sid], buf1k)
        pltpu.sync_copy(pi_hbm.at[sid], pi_v)
        pltpu.sync_copy(pf_hbm.at[sid], pf_v)
        kr = pi_v[pl.ds(0, 16)]
        pvec = pf_v[pl.ds(0, 16)]
        k1 = pi_v[pl.ds(16, 16)][0]
        y0 = pf_v[pl.ds(16, 16)][0]
        base = lane * HS

        @plsc.parallel_loop(0, HS, unroll=8, carry=(zeros_f, zeros_i, top_v))
        def _init(j, c):
            tv, mx, mn = c
            wb = plsc.load_gather(buf1k, [base + j])
            bufb[pl.ds(j * 16, 16)] = wb
            return (tv + lax.bitcast_convert_type(wb, jnp.float32),
                    jnp.maximum(mx, wb), jnp.minimum(mn, wb))

        tot, mxv, mnv = _init

        def unconverged(carry):
            lo, hi = carry
            return jnp.max(hi - lo) > 0

        def qpass(carry):
            lo, hi = carry
            mid = lo + lax.shift_right_logical(hi - lo, 1)

            @plsc.parallel_loop(0, HS, unroll=8, carry=zeros_i)
            def _cnt(j, cnt):
                wb = bufb[pl.ds(j * 16, 16)]
                return cnt + jnp.where(wb <= mid, 1, 0).astype(jnp.int32)

            ge = _cnt >= kr
            return jnp.where(ge, lo, mid + 1), jnp.where(ge, mid, hi)

        lo, _ = lax.while_loop(unconverged, qpass, (mnv, mxv))
        t = lax.bitcast_convert_type(lo, jnp.float32)

        @plsc.parallel_loop(0, HS, unroll=8, carry=(zeros_i, zeros_f))
        def _fin(j, c):
            clt, slt = c
            wb = bufb[pl.ds(j * 16, 16)]
            m = wb < lo
            return (clt + jnp.where(m, 1, 0).astype(jnp.int32),
                    slt + jnp.where(
                        m, lax.bitcast_convert_type(wb, jnp.float32), 0.0))

        cltv, sltv = _fin
        res = jnp.where(
            kr >= HS, tot,
            jnp.where(kr <= 0, 0.0,
                      sltv + (kr - cltv).astype(jnp.float32) * t))
        rres = jnp.sum(pvec * res)
        sk, _ = plsc.sort_key_val(tot, tot)
        a_val = jnp.sum(jnp.where(lane < k1, sk, 0.0))
        outv[...] = jnp.where(lane == 0, y0 * a_val + rres, 0.0)

    pltpu.sync_copy(outv, out_hbm.at[wid])


def _sc_select(colsq, sc3, pi, pf):
    mesh = plsc.VectorSubcoreMesh(core_axis_name="c", subcore_axis_name="s")
    run = functools.partial(
        pl.kernel,
        mesh=mesh,
        compiler_params=pltpu.CompilerParams(needs_layout_passes=False),
        out_type=jax.ShapeDtypeStruct((32, 16), jnp.float32),
        scratch_types=[
            pltpu.VMEM((IN3 + 16,), jnp.int32),
            pltpu.VMEM((IN3 + 16,), jnp.int32),
            pltpu.VMEM((IN1,), jnp.int32),
            pltpu.VMEM((32,), jnp.int32),
            pltpu.VMEM((32,), jnp.float32),
            pltpu.VMEM((16,), jnp.float32),
        ],
    )(_sc_body)
    return run(colsq, sc3, pi, pf)


def kernel(W1, W3, s, r, y, p):
    colsq3, sc33 = _tc_reduce(W1, W3)
    colsq = lax.bitcast_convert_type(colsq3.reshape(L, IN1), jnp.int32)
    sc3 = lax.bitcast_convert_type(sc33.reshape(L, IN3), jnp.int32)
    k1 = jnp.ceil(s[:, 0]).astype(jnp.int32)
    k2 = jnp.ceil(s[:, 1]).astype(jnp.int32)
    kr = jnp.ceil(r).astype(jnp.int32)
    pi = jnp.concatenate(
        [kr, k1[:, None], k2[:, None], jnp.zeros((L, 14), jnp.int32)], axis=1)
    pf = jnp.concatenate([p, y, jnp.zeros((L, 14), jnp.float32)], axis=1)
    parts = _sc_select(colsq, sc3, pi, pf)
    return jnp.sum(parts)
